# Initial kernel scaffold; baseline (speedup 1.0000x reference)
import jax
import jax.numpy as jnp
from jax.experimental import pallas as pl


def kernel(x, position_ids, cos_cached, sin_cached):
    raise NotImplementedError("write your pallas kernel here")


# speedup vs baseline: 3.7694x; 3.7694x over previous
import functools

import jax
import jax.numpy as jnp
from jax import lax
from jax.experimental import pallas as pl
from jax.experimental.pallas import tpu as pltpu
from jax.experimental.pallas import tpu_sc as plsc

_NC, _NS = 2, 16
_NW = _NC * _NS
_B = 4 * 4096
_BW = _B // _NW
_D = 128

_mesh = plsc.VectorSubcoreMesh(core_axis_name="c", subcore_axis_name="s")


@functools.partial(
    pl.kernel,
    out_type=(
        jax.ShapeDtypeStruct((_B, _D), jnp.float32),
        jax.ShapeDtypeStruct((_B, _D), jnp.float32),
    ),
    mesh=_mesh,
    scratch_types=[
        pltpu.VMEM((_BW,), jnp.int32),
        pltpu.VMEM((_BW, _D), jnp.float32),
        pltpu.SemaphoreType.DMA,
    ],
)
def _rope_gather(cos_hbm, sin_hbm, idx_hbm, cos_out, sin_out,
                 idx_v, rows_v, sem):
    wid = lax.axis_index("s") * _NC + lax.axis_index("c")
    base = wid * _BW
    pltpu.sync_copy(idx_hbm.at[pl.ds(base, _BW)], idx_v)
    pltpu.async_copy(cos_hbm.at[idx_v], rows_v, sem).wait()
    pltpu.sync_copy(rows_v, cos_out.at[pl.ds(base, _BW)])
    pltpu.async_copy(sin_hbm.at[idx_v], rows_v, sem).wait()
    pltpu.sync_copy(rows_v, sin_out.at[pl.ds(base, _BW)])


def kernel(x, position_ids, cos_cached, sin_cached):
    bsz, seq = position_ids.shape
    idx = position_ids.reshape(-1).astype(jnp.int32)
    cos_flat, sin_flat = _rope_gather(cos_cached, sin_cached, idx)
    cos = cos_flat.reshape(bsz, seq, _D).astype(x.dtype)
    sin = sin_flat.reshape(bsz, seq, _D).astype(x.dtype)
    return (cos, sin)
